# baseline (speedup 1.0000x reference)
import functools

import jax
import jax.numpy as jnp
from jax import lax
from jax.experimental import pallas as pl
from jax.experimental.pallas import tpu as pltpu
from jax.experimental.pallas import tpu_sc as plsc

_LANES = 16


def _gather_sc(inputs, tableT):
    D, V = tableT.shape
    B = inputs.shape[0]
    info = plsc.get_sparse_core_info()
    nw = info.num_cores * info.num_subcores
    b_per_w = B // nw
    n_dgroups = D // _LANES
    mesh = plsc.VectorSubcoreMesh(core_axis_name="c", subcore_axis_name="s")

    @functools.partial(
        pl.kernel,
        mesh=mesh,
        compiler_params=pltpu.CompilerParams(needs_layout_passes=False),
        out_type=jax.ShapeDtypeStruct((B, D), jnp.float32),
        scratch_types=[
            pltpu.VMEM((b_per_w,), jnp.int32),
            pltpu.VMEM((D, 128), jnp.float32),
            pltpu.VMEM((D, 128), jnp.float32),
            pltpu.VMEM((D, 128), jnp.float32),
            pltpu.VMEM((D, 128), jnp.float32),
            pltpu.VMEM((D, 128), jnp.float32),
            pltpu.VMEM((D, 128), jnp.float32),
            pltpu.VMEM((D, 128), jnp.float32),
            pltpu.VMEM((D, 128), jnp.float32),
            pltpu.VMEM((b_per_w, D), jnp.float32),
            pltpu.SemaphoreType.DMA,
            pltpu.SemaphoreType.DMA,
            pltpu.SemaphoreType.DMA,
            pltpu.SemaphoreType.DMA,
            pltpu.SemaphoreType.DMA,
            pltpu.SemaphoreType.DMA,
            pltpu.SemaphoreType.DMA,
            pltpu.SemaphoreType.DMA,
        ],
    )
    def gather_kernel(
        idx_hbm, table_hbm, out_hbm, idx_v,
        buf0_v, buf1_v, buf2_v, buf3_v,
        buf4_v, buf5_v, buf6_v, buf7_v, emb_v,
        sem0, sem1, sem2, sem3, sem4, sem5, sem6, sem7,
    ):
        wid = lax.axis_index("s") * info.num_cores + lax.axis_index("c")
        base = wid * b_per_w
        pltpu.sync_copy(idx_hbm.at[pl.ds(base, b_per_w)], idx_v)
        sems = (sem0, sem1, sem2, sem3, sem4, sem5, sem6, sem7)
        bufs = (buf0_v, buf1_v, buf2_v, buf3_v, buf4_v, buf5_v, buf6_v, buf7_v)
        nbuf = 8

        def fetch(b):
            vec = idx_v[pl.ds((b // _LANES) * _LANES, _LANES)]
            i = vec[b % _LANES]
            col = lax.div(i, 128)
            return pltpu.make_async_copy(
                table_hbm.at[:, pl.ds(col * 128, 128)],
                bufs[b % nbuf],
                sems[b % nbuf],
            ), lax.rem(i, 128)

        copies = []
        lanes = []
        for b in range(nbuf - 1):
            cp, lane = fetch(b)
            cp.start()
            copies.append(cp)
            lanes.append(lane)
        for b in range(b_per_w):
            if b + nbuf - 1 < b_per_w:
                cp_n, lane_n = fetch(b + nbuf - 1)
                cp_n.start()
                copies.append(cp_n)
                lanes.append(lane_n)
            copies[b].wait()
            lane_vec = jnp.full((_LANES,), lanes[b], jnp.int32)
            for g in range(n_dgroups):
                d_vec = lax.iota(jnp.int32, _LANES) + g * _LANES
                vals = plsc.load_gather(bufs[b % nbuf], [d_vec, lane_vec])
                emb_v[b, pl.ds(g * _LANES, _LANES)] = vals
        pltpu.sync_copy(emb_v, out_hbm.at[pl.ds(base, b_per_w)])

    return gather_kernel(inputs, tableT)


def _project_tc(embeds, Wt, b, vt=4608):
    B, D = embeds.shape
    V = Wt.shape[1]
    grid = pl.cdiv(V, vt)
    b2 = b.reshape(1, V)

    def mm(e_ref, w_ref, b_ref, o_ref):
        acc = lax.dot_general(
            w_ref[...], e_ref[...],
            dimension_numbers=(((0,), (1,)), ((), ())),
            preferred_element_type=jnp.float32,
        )
        ones = jnp.ones((1, B), jnp.float32)
        bias = lax.dot_general(
            b_ref[...], ones,
            dimension_numbers=(((0,), (0,)), ((), ())),
            preferred_element_type=jnp.float32,
        )
        o_ref[...] = acc + bias

    return pl.pallas_call(
        mm,
        grid=(grid,),
        in_specs=[
            pl.BlockSpec((B, D), lambda j: (0, 0)),
            pl.BlockSpec((D, vt), lambda j: (0, j)),
            pl.BlockSpec((1, vt), lambda j: (0, j)),
        ],
        out_specs=pl.BlockSpec((vt, B), lambda j: (j, 0)),
        out_shape=jax.ShapeDtypeStruct((V, B), jnp.float32),
    )(embeds, Wt, b2)


def kernel(inputs, emb_table, W, b):
    tableT = emb_table.T
    Wt = W.T
    embeds = _gather_sc(inputs, tableT)
    outT = _project_tc(embeds, Wt, b)
    return outT.T


# speedup vs baseline: 1.0070x; 1.0070x over previous
import functools

import jax
import jax.numpy as jnp
from jax import lax
from jax.experimental import pallas as pl
from jax.experimental.pallas import tpu as pltpu
from jax.experimental.pallas import tpu_sc as plsc

_LANES = 16


def _gather_sc(inputs, tableT):
    D, V = tableT.shape
    B = inputs.shape[0]
    info = plsc.get_sparse_core_info()
    nw = info.num_cores * info.num_subcores
    b_per_w = B // nw
    n_dgroups = D // _LANES
    mesh = plsc.VectorSubcoreMesh(core_axis_name="c", subcore_axis_name="s")

    @functools.partial(
        pl.kernel,
        mesh=mesh,
        compiler_params=pltpu.CompilerParams(needs_layout_passes=False),
        out_type=jax.ShapeDtypeStruct((B, D), jnp.float32),
        scratch_types=[
            pltpu.VMEM((b_per_w,), jnp.int32),
            pltpu.VMEM((D, 128), jnp.float32),
            pltpu.VMEM((D, 128), jnp.float32),
            pltpu.VMEM((D, 128), jnp.float32),
            pltpu.VMEM((D, 128), jnp.float32),
            pltpu.VMEM((D, 128), jnp.float32),
            pltpu.VMEM((D, 128), jnp.float32),
            pltpu.VMEM((D, 128), jnp.float32),
            pltpu.VMEM((D, 128), jnp.float32),
            pltpu.VMEM((b_per_w, D), jnp.float32),
            pltpu.SemaphoreType.DMA,
            pltpu.SemaphoreType.DMA,
            pltpu.SemaphoreType.DMA,
            pltpu.SemaphoreType.DMA,
            pltpu.SemaphoreType.DMA,
            pltpu.SemaphoreType.DMA,
            pltpu.SemaphoreType.DMA,
            pltpu.SemaphoreType.DMA,
        ],
    )
    def gather_kernel(
        idx_hbm, table_hbm, out_hbm, idx_v,
        buf0_v, buf1_v, buf2_v, buf3_v,
        buf4_v, buf5_v, buf6_v, buf7_v, emb_v,
        sem0, sem1, sem2, sem3, sem4, sem5, sem6, sem7,
    ):
        wid = lax.axis_index("s") * info.num_cores + lax.axis_index("c")
        base = wid * b_per_w
        pltpu.sync_copy(idx_hbm.at[pl.ds(base, b_per_w)], idx_v)
        sems = (sem0, sem1, sem2, sem3, sem4, sem5, sem6, sem7)
        bufs = (buf0_v, buf1_v, buf2_v, buf3_v, buf4_v, buf5_v, buf6_v, buf7_v)
        nbuf = 8

        def fetch(b):
            vec = idx_v[pl.ds((b // _LANES) * _LANES, _LANES)]
            i = vec[b % _LANES]
            col = lax.div(i, 128)
            return pltpu.make_async_copy(
                table_hbm.at[:, pl.ds(col * 128, 128)],
                bufs[b % nbuf],
                sems[b % nbuf],
            ), lax.rem(i, 128)

        copies = []
        lanes = []
        for b in range(nbuf - 1):
            cp, lane = fetch(b)
            cp.start()
            copies.append(cp)
            lanes.append(lane)
        for b in range(b_per_w):
            if b + nbuf - 1 < b_per_w:
                cp_n, lane_n = fetch(b + nbuf - 1)
                cp_n.start()
                copies.append(cp_n)
                lanes.append(lane_n)
            copies[b].wait()
            lane_vec = jnp.full((_LANES,), lanes[b], jnp.int32)
            for g in range(n_dgroups):
                d_vec = lax.iota(jnp.int32, _LANES) + g * _LANES
                vals = plsc.load_gather(bufs[b % nbuf], [d_vec, lane_vec])
                emb_v[b, pl.ds(g * _LANES, _LANES)] = vals
        pltpu.sync_copy(emb_v, out_hbm.at[pl.ds(base, b_per_w)])

    return gather_kernel(inputs, tableT)


def _project_tc(embeds, Wt, b, vt=4096):
    B, D = embeds.shape
    V = Wt.shape[1]
    grid = pl.cdiv(V, vt)
    b2 = b.reshape(1, V)

    def mm(e_ref, w_ref, b_ref, o_ref):
        acc = lax.dot_general(
            w_ref[...], e_ref[...],
            dimension_numbers=(((0,), (1,)), ((), ())),
            preferred_element_type=jnp.float32,
        )
        ones = jnp.ones((1, B), jnp.float32)
        bias = lax.dot_general(
            b_ref[...], ones,
            dimension_numbers=(((0,), (0,)), ((), ())),
            preferred_element_type=jnp.float32,
        )
        o_ref[...] = acc + bias

    return pl.pallas_call(
        mm,
        grid=(grid,),
        in_specs=[
            pl.BlockSpec((B, D), lambda j: (0, 0)),
            pl.BlockSpec((D, vt), lambda j: (0, j)),
            pl.BlockSpec((1, vt), lambda j: (0, j)),
        ],
        out_specs=pl.BlockSpec((vt, B), lambda j: (j, 0)),
        out_shape=jax.ShapeDtypeStruct((V, B), jnp.float32),
    )(embeds, Wt, b2)


def kernel(inputs, emb_table, W, b):
    tableT = emb_table.T
    Wt = W.T
    embeds = _gather_sc(inputs, tableT)
    outT = _project_tc(embeds, Wt, b)
    return outT.T
